# baseline (speedup 1.0000x reference)
import jax
import jax.numpy as jnp
from jax import lax
from jax.experimental import pallas as pl
from jax.experimental.pallas import tpu as pltpu
from jax.experimental.pallas import tpu_sc as plsc

N = 100000
E = 3200000
DIM = 128
NC = 2
NS = 16
NW = NC * NS
NPAD = 100352
SLICE = NPAD // NS
EPW = E // NW
CH = 4000
CH1 = 25000
NCHUNK = EPW // CH
RBLK = 1024
NBLK = NPAD // RBLK

_SC_PARAMS = pltpu.CompilerParams(use_tc_tiling_on_sc=False)


def _k1_body(ei_hbm, z1_hbm, ones_hbm, ho_hbm,
             ones_v, idx0, idx1, zbuf, ho_sh, hi_sh, sem0, sem1):
    c = lax.axis_index("c")
    s = lax.axis_index("s")
    w = c * NS + s
    pltpu.sync_copy(z1_hbm.at[pl.ds(0, SLICE)], zbuf)
    pltpu.sync_copy(zbuf, ho_sh.at[pl.ds(s * SLICE, SLICE)])
    pltpu.sync_copy(zbuf, hi_sh.at[pl.ds(s * SLICE, SLICE)])
    pltpu.sync_copy(ones_hbm, ones_v)
    plsc.subcore_barrier()
    base = w * EPW

    idx = (idx0, idx1)
    sems = (sem0, sem1)
    hs = (ho_sh, hi_sh)
    sd = [None, None]
    for k in range(2 * (EPW // CH1)):
        b = k % 2
        i = k // 2
        which = k % 2
        if sd[b] is not None:
            sd[b].wait()
            sd[b] = None
        off = base + i * CH1 + (E if which else 0)
        pltpu.sync_copy(ei_hbm.at[pl.ds(off, CH1)], idx[b])
        sd[b] = pltpu.async_copy(ones_v, hs[which].at[idx[b]], sems[b],
                                 add=True)
    for b in range(2):
        if sd[b] is not None:
            sd[b].wait()
    plsc.subcore_barrier()
    pltpu.sync_copy(ho_sh.at[pl.ds(s * SLICE, SLICE)], zbuf)
    pltpu.sync_copy(zbuf, ho_hbm.at[pl.ds(c * 2 * NPAD + s * SLICE, SLICE)])
    pltpu.sync_copy(hi_sh.at[pl.ds(s * SLICE, SLICE)], zbuf)
    pltpu.sync_copy(zbuf, ho_hbm.at[pl.ds(c * 2 * NPAD + NPAD + s * SLICE,
                                          SLICE)])


def _hist(ei, z1, ones_c):
    mesh = plsc.VectorSubcoreMesh(core_axis_name="c", subcore_axis_name="s")
    f = pl.kernel(
        _k1_body,
        out_type=jax.ShapeDtypeStruct((NC * 2 * NPAD,), jnp.float32),
        mesh=mesh,
        scratch_types=[
            pltpu.VMEM((CH1,), jnp.float32),
            pltpu.VMEM((CH1,), jnp.int32),
            pltpu.VMEM((CH1,), jnp.int32),
            pltpu.VMEM((SLICE,), jnp.float32),
            pltpu.VMEM_SHARED((NPAD,), jnp.float32),
            pltpu.VMEM_SHARED((NPAD,), jnp.float32),
            pltpu.SemaphoreType.DMA,
            pltpu.SemaphoreType.DMA,
        ],
        compiler_params=_SC_PARAMS,
    )
    return f(ei, z1, ones_c)


def _k3_body(ei_hbm, w_hbm, z1_hbm, zi1_hbm,
             oq_hbm, osx_hbm,
             sidx0, sidx1, didx0, didx1,
             wb0, wb1, qb0, qb1, xb0, xb1, zbuf, zbufi,
             aq_sh, asx_sh,
             semg0, semg1, sems0, sems1):
    c = lax.axis_index("c")
    s = lax.axis_index("s")
    w = c * NS + s
    pltpu.sync_copy(z1_hbm.at[pl.ds(0, SLICE)], zbuf)
    pltpu.sync_copy(zi1_hbm.at[pl.ds(0, SLICE)], zbufi)
    pltpu.sync_copy(zbuf, aq_sh.at[pl.ds(s * SLICE, SLICE)])
    pltpu.sync_copy(zbufi, asx_sh.at[pl.ds(s * SLICE, SLICE)])
    plsc.subcore_barrier()
    base = w * EPW

    sidx = (sidx0, sidx1)
    didx = (didx0, didx1)
    wb = (wb0, wb1)
    qb = (qb0, qb1)
    xb = (xb0, xb1)
    semg = (semg0, semg1)
    sems = (sems0, sems1)
    gd = [None, None]
    sd = [None, None]
    m16 = jnp.int32(0xFFFF)
    mhi = jnp.int32(-65536)

    def unpack(b):
        def step(k, carry):
            ww = wb[b][pl.ds(k * 16, 16)]
            qb[b][pl.ds(k * 16, 16)] = lax.bitcast_convert_type(
                ww & mhi, jnp.float32)
            t = ww & m16
            sq = lax.shift_right_logical(t, 1)
            xq = sq * (t & 1)
            xb[b][pl.ds(k * 16, 16)] = lax.shift_left(sq, 16) | xq
            return carry
        lax.fori_loop(0, CH // 16, step, 0)

    for i in range(NCHUNK + 1):
        b = i % 2
        pb = 1 - b
        if i < NCHUNK:
            if sd[b] is not None:
                for dd in sd[b]:
                    dd.wait()
                sd[b] = None
            off = base + i * CH
            pltpu.sync_copy(ei_hbm.at[pl.ds(off, CH)], sidx[b])
            pltpu.sync_copy(ei_hbm.at[pl.ds(E + off, CH)], didx[b])
            gd[b] = pltpu.async_copy(w_hbm.at[sidx[b]], wb[b], semg[b])
        if i > 0:
            gd[pb].wait()
            gd[pb] = None
            unpack(pb)
            sd[pb] = [
                pltpu.async_copy(qb[pb], aq_sh.at[didx[pb]], sems[pb],
                                 add=True),
                pltpu.async_copy(xb[pb], asx_sh.at[didx[pb]], sems[pb],
                                 add=True),
            ]
    for b in range(2):
        if sd[b] is not None:
            for dd in sd[b]:
                dd.wait()

    plsc.subcore_barrier()
    pltpu.sync_copy(aq_sh.at[pl.ds(s * SLICE, SLICE)], zbuf)
    pltpu.sync_copy(zbuf, oq_hbm.at[pl.ds(c * NPAD + s * SLICE, SLICE)])
    pltpu.sync_copy(asx_sh.at[pl.ds(s * SLICE, SLICE)], zbufi)
    pltpu.sync_copy(zbufi, osx_hbm.at[pl.ds(c * NPAD + s * SLICE, SLICE)])


def _scatter(ei, wtbl, z1, zi1):
    mesh = plsc.VectorSubcoreMesh(core_axis_name="c", subcore_axis_name="s")
    f = pl.kernel(
        _k3_body,
        out_type=(jax.ShapeDtypeStruct((NC * NPAD,), jnp.float32),
                  jax.ShapeDtypeStruct((NC * NPAD,), jnp.int32)),
        mesh=mesh,
        scratch_types=(
            [pltpu.VMEM((CH,), jnp.int32)] * 4
            + [pltpu.VMEM((CH,), jnp.int32)] * 2
            + [pltpu.VMEM((CH,), jnp.float32)] * 2
            + [pltpu.VMEM((CH,), jnp.int32)] * 2
            + [pltpu.VMEM((SLICE,), jnp.float32),
               pltpu.VMEM((SLICE,), jnp.int32)]
            + [pltpu.VMEM_SHARED((NPAD,), jnp.float32),
               pltpu.VMEM_SHARED((NPAD,), jnp.int32)]
            + [pltpu.SemaphoreType.DMA] * 4
        ),
        compiler_params=_SC_PARAMS,
    )
    return f(ei, wtbl, z1, zi1)


SQ = 512


def _k2_body(ho0_ref, ho1_ref, nw_ref, sg_ref, w_ref):
    od = jnp.maximum(ho0_ref[...] + ho1_ref[...], 1.0)
    sc = lax.rsqrt(od)
    q = nw_ref[...] * sc
    qb = lax.bitcast_convert_type(q, jnp.int32)
    qr = (qb + 0x7FFF + (lax.shift_right_logical(qb, 16) & 1)) & ~0xFFFF
    s_q = (sc * SQ + 0.5).astype(jnp.int32)
    sig = sg_ref[...].astype(jnp.int32)
    w_ref[...] = qr | lax.shift_left(s_q, 1) | sig


ROWS = NPAD // DIM


def _table(ho0, ho1, nwp, sgp):
    full = pl.BlockSpec((ROWS, DIM), lambda i: (0, 0))
    return pl.pallas_call(
        _k2_body,
        grid=(1,),
        in_specs=[full, full, full, full],
        out_specs=full,
        out_shape=jax.ShapeDtypeStruct((ROWS, DIM), jnp.int32),
    )(ho0, ho1, nwp, sgp)


PACK = 8


def _k4_body(q0_ref, q1_ref, sx0_ref, sx1_ref, c0_ref, c1_ref,
             emb_ref, w_ref, b_ref, o_ref):
    q = q0_ref[...] + q1_ref[...]
    cnt = c0_ref[...] + c1_ref[...]
    sxw = sx0_ref[...] + sx1_ref[...]
    m16 = jnp.int32(0xFFFF)
    inv = jnp.float32(1.0 / SQ)
    S = (lax.shift_right_logical(sxw, 16) & m16).astype(jnp.float32) * inv
    X = (sxw & m16).astype(jnp.float32) * inv
    e00 = emb_ref[0, 0]
    e01 = emb_ref[0, 1]
    e10 = emb_ref[1, 0]
    e11 = emb_ref[1, 1]
    f1 = e00 * S + (e10 - e00) * X
    f2 = e01 * S + (e11 - e01) * X
    r = lax.rsqrt(jnp.maximum(cnt, 1.0))
    w = w_ref[...]
    t = ((q * r)[:, :, None] * w[0].reshape(1, 1, DIM)
         + (f1 * r)[:, :, None] * w[1].reshape(1, 1, DIM)
         + (f2 * r)[:, :, None] * w[2].reshape(1, 1, DIM)
         + b_ref[...].reshape(1, 1, DIM))
    o_ref[...] = t.reshape(RBLK, DIM)


def _project(q0, q1, sx0, sx1, c0, c1, emb, W0, b0r):
    pk = pl.BlockSpec((PACK, DIM), lambda i: (i, 0))
    return pl.pallas_call(
        _k4_body,
        grid=(NBLK,),
        in_specs=[pk] * 6 + [pl.BlockSpec((2, 2), lambda i: (0, 0)),
                             pl.BlockSpec((3, DIM), lambda i: (0, 0)),
                             pl.BlockSpec((1, DIM), lambda i: (0, 0))],
        out_specs=pl.BlockSpec((RBLK, DIM), lambda i: (i, 0)),
        out_shape=jax.ShapeDtypeStruct((N, DIM), jnp.float32),
    )(q0, q1, sx0, sx1, c0, c1, emb, W0, b0r)


def kernel(significance, node_weight, edge_index, emb_table, W0, b0):
    ei = edge_index.astype(jnp.int32).reshape(2 * E)
    nwp = jnp.pad(node_weight.astype(jnp.float32), (0, NPAD - N)).reshape(ROWS, DIM)
    sgp = jnp.pad(significance.astype(jnp.float32), (0, NPAD - N)).reshape(ROWS, DIM)
    z1 = jnp.zeros((NPAD,), jnp.float32)
    zi1 = jnp.zeros((NPAD,), jnp.int32)
    ones_c1 = jnp.ones((CH1,), jnp.float32)
    emb = emb_table.astype(jnp.float32)

    ho = _hist(ei, z1, ones_c1)
    h4 = ho.reshape(NC, 2, ROWS, DIM)
    wtbl = _table(h4[0, 0], h4[1, 0], nwp, sgp)
    oq, osx = _scatter(ei, wtbl.reshape(NPAD), z1, zi1)
    qp = oq.reshape(NC, ROWS, DIM)
    sxp = osx.reshape(NC, ROWS, DIM)
    cp = h4[:, 1]
    out = _project(qp[0], qp[1], sxp[0], sxp[1], cp[0], cp[1], emb,
                   W0.astype(jnp.float32),
                   b0.astype(jnp.float32).reshape(1, DIM))
    return out


# speedup vs baseline: 1.0837x; 1.0837x over previous
import jax
import jax.numpy as jnp
from jax import lax
from jax.experimental import pallas as pl
from jax.experimental.pallas import tpu as pltpu
from jax.experimental.pallas import tpu_sc as plsc

N = 100000
E = 3200000
DIM = 128
NC = 2
NS = 16
NW = NC * NS
NPAD = 100352
SLICE = NPAD // NS
EPW = E // NW
CH = 4000
CH1 = 50000
NCHUNK = EPW // CH
RBLK = 1024
NBLK = NPAD // RBLK

_SC_PARAMS = pltpu.CompilerParams(use_tc_tiling_on_sc=False)


def _k1_body(ei_hbm, z1_hbm, ones_hbm, ho_hbm, ones_v, idx_v, zbuf, hist_sh):
    c = lax.axis_index("c")
    s = lax.axis_index("s")
    w = c * NS + s
    pltpu.sync_copy(z1_hbm.at[pl.ds(0, SLICE)], zbuf)
    pltpu.sync_copy(zbuf, hist_sh.at[pl.ds(s * SLICE, SLICE)])
    pltpu.sync_copy(ones_hbm, ones_v)
    plsc.subcore_barrier()
    base = w * EPW

    def chunk(i, carry):
        pltpu.sync_copy(ei_hbm.at[pl.ds(base + i * CH1, CH1)], idx_v)
        pltpu.sync_copy(ones_v, hist_sh.at[idx_v], add=True)
        return carry

    lax.fori_loop(0, EPW // CH1, chunk, 0)
    plsc.subcore_barrier()
    pltpu.sync_copy(hist_sh.at[pl.ds(s * SLICE, SLICE)], zbuf)
    pltpu.sync_copy(zbuf, ho_hbm.at[pl.ds(c * NPAD + s * SLICE, SLICE)])


def _hist(ei, z1, ones_c):
    mesh = plsc.VectorSubcoreMesh(core_axis_name="c", subcore_axis_name="s")
    f = pl.kernel(
        _k1_body,
        out_type=jax.ShapeDtypeStruct((NC * NPAD,), jnp.float32),
        mesh=mesh,
        scratch_types=[
            pltpu.VMEM((CH1,), jnp.float32),
            pltpu.VMEM((CH1,), jnp.int32),
            pltpu.VMEM((SLICE,), jnp.float32),
            pltpu.VMEM_SHARED((NPAD,), jnp.float32),
        ],
        compiler_params=_SC_PARAMS,
    )
    return f(ei, z1, ones_c)


def _k3_body(ei_hbm, w_hbm, z1_hbm, zi1_hbm, ones_hbm,
             oq_hbm, osx_hbm, oc_hbm,
             sidx0, sidx1, didx0, didx1,
             wb0, wb1, qb0, qb1, xb0, xb1, ones_v, zbuf, zbufi,
             aq_sh, asx_sh, ac_sh,
             semg0, semg1, sems0, sems1):
    c = lax.axis_index("c")
    s = lax.axis_index("s")
    w = c * NS + s
    pltpu.sync_copy(z1_hbm.at[pl.ds(0, SLICE)], zbuf)
    pltpu.sync_copy(zi1_hbm.at[pl.ds(0, SLICE)], zbufi)
    pltpu.sync_copy(zbuf, aq_sh.at[pl.ds(s * SLICE, SLICE)])
    pltpu.sync_copy(zbuf, ac_sh.at[pl.ds(s * SLICE, SLICE)])
    pltpu.sync_copy(zbufi, asx_sh.at[pl.ds(s * SLICE, SLICE)])
    pltpu.sync_copy(ones_hbm, ones_v)
    plsc.subcore_barrier()
    base = w * EPW

    sidx = (sidx0, sidx1)
    didx = (didx0, didx1)
    wb = (wb0, wb1)
    qb = (qb0, qb1)
    xb = (xb0, xb1)
    semg = (semg0, semg1)
    sems = (sems0, sems1)
    gd = [None, None]
    sd = [None, None]
    m16 = jnp.int32(0xFFFF)
    mhi = jnp.int32(-65536)

    def unpack(b):
        def step(k, carry):
            ww = wb[b][pl.ds(k * 16, 16)]
            qb[b][pl.ds(k * 16, 16)] = lax.bitcast_convert_type(
                ww & mhi, jnp.float32)
            t = ww & m16
            sq = lax.shift_right_logical(t, 1)
            xq = sq * (t & 1)
            xb[b][pl.ds(k * 16, 16)] = lax.shift_left(sq, 16) | xq
            return carry
        lax.fori_loop(0, CH // 16, step, 0)

    for i in range(NCHUNK + 1):
        b = i % 2
        pb = 1 - b
        if i < NCHUNK:
            if sd[b] is not None:
                for dd in sd[b]:
                    dd.wait()
                sd[b] = None
            off = base + i * CH
            pltpu.sync_copy(ei_hbm.at[pl.ds(off, CH)], sidx[b])
            pltpu.sync_copy(ei_hbm.at[pl.ds(E + off, CH)], didx[b])
            gd[b] = pltpu.async_copy(w_hbm.at[sidx[b]], wb[b], semg[b])
        if i > 0:
            gd[pb].wait()
            gd[pb] = None
            unpack(pb)
            sd[pb] = [
                pltpu.async_copy(qb[pb], aq_sh.at[didx[pb]], sems[pb],
                                 add=True),
                pltpu.async_copy(xb[pb], asx_sh.at[didx[pb]], sems[pb],
                                 add=True),
                pltpu.async_copy(ones_v, ac_sh.at[didx[pb]], sems[pb],
                                 add=True),
            ]
    for b in range(2):
        if sd[b] is not None:
            for dd in sd[b]:
                dd.wait()

    plsc.subcore_barrier()
    pltpu.sync_copy(aq_sh.at[pl.ds(s * SLICE, SLICE)], zbuf)
    pltpu.sync_copy(zbuf, oq_hbm.at[pl.ds(c * NPAD + s * SLICE, SLICE)])
    pltpu.sync_copy(ac_sh.at[pl.ds(s * SLICE, SLICE)], zbuf)
    pltpu.sync_copy(zbuf, oc_hbm.at[pl.ds(c * NPAD + s * SLICE, SLICE)])
    pltpu.sync_copy(asx_sh.at[pl.ds(s * SLICE, SLICE)], zbufi)
    pltpu.sync_copy(zbufi, osx_hbm.at[pl.ds(c * NPAD + s * SLICE, SLICE)])


def _scatter(ei, wtbl, z1, zi1, ones_c):
    mesh = plsc.VectorSubcoreMesh(core_axis_name="c", subcore_axis_name="s")
    f = pl.kernel(
        _k3_body,
        out_type=(jax.ShapeDtypeStruct((NC * NPAD,), jnp.float32),
                  jax.ShapeDtypeStruct((NC * NPAD,), jnp.int32),
                  jax.ShapeDtypeStruct((NC * NPAD,), jnp.float32)),
        mesh=mesh,
        scratch_types=(
            [pltpu.VMEM((CH,), jnp.int32)] * 4
            + [pltpu.VMEM((CH,), jnp.int32)] * 2
            + [pltpu.VMEM((CH,), jnp.float32)] * 2
            + [pltpu.VMEM((CH,), jnp.int32)] * 2
            + [pltpu.VMEM((CH,), jnp.float32)]
            + [pltpu.VMEM((SLICE,), jnp.float32),
               pltpu.VMEM((SLICE,), jnp.int32)]
            + [pltpu.VMEM_SHARED((NPAD,), jnp.float32),
               pltpu.VMEM_SHARED((NPAD,), jnp.int32),
               pltpu.VMEM_SHARED((NPAD,), jnp.float32)]
            + [pltpu.SemaphoreType.DMA] * 4
        ),
        compiler_params=_SC_PARAMS,
    )
    return f(ei, wtbl, z1, zi1, ones_c)


SQ = 512


def _k2_body(ho0_ref, ho1_ref, nw_ref, sg_ref, w_ref):
    od = jnp.maximum(ho0_ref[...] + ho1_ref[...], 1.0)
    sc = lax.rsqrt(od)
    q = nw_ref[...] * sc
    qb = lax.bitcast_convert_type(q, jnp.int32)
    qr = (qb + 0x7FFF + (lax.shift_right_logical(qb, 16) & 1)) & ~0xFFFF
    s_q = (sc * SQ + 0.5).astype(jnp.int32)
    sig = sg_ref[...].astype(jnp.int32)
    w_ref[...] = qr | lax.shift_left(s_q, 1) | sig


ROWS = NPAD // DIM


def _table(ho0, ho1, nwp, sgp):
    full = pl.BlockSpec((ROWS, DIM), lambda i: (0, 0))
    return pl.pallas_call(
        _k2_body,
        grid=(1,),
        in_specs=[full, full, full, full],
        out_specs=full,
        out_shape=jax.ShapeDtypeStruct((ROWS, DIM), jnp.int32),
    )(ho0, ho1, nwp, sgp)


PACK = 8


def _k4_body(q0_ref, q1_ref, sx0_ref, sx1_ref, c0_ref, c1_ref,
             emb_ref, w_ref, b_ref, o_ref):
    q = q0_ref[...] + q1_ref[...]
    cnt = c0_ref[...] + c1_ref[...]
    sxw = sx0_ref[...] + sx1_ref[...]
    m16 = jnp.int32(0xFFFF)
    inv = jnp.float32(1.0 / SQ)
    S = (lax.shift_right_logical(sxw, 16) & m16).astype(jnp.float32) * inv
    X = (sxw & m16).astype(jnp.float32) * inv
    e00 = emb_ref[0, 0]
    e01 = emb_ref[0, 1]
    e10 = emb_ref[1, 0]
    e11 = emb_ref[1, 1]
    f1 = e00 * S + (e10 - e00) * X
    f2 = e01 * S + (e11 - e01) * X
    r = lax.rsqrt(jnp.maximum(cnt, 1.0))
    w = w_ref[...]
    t = ((q * r)[:, :, None] * w[0].reshape(1, 1, DIM)
         + (f1 * r)[:, :, None] * w[1].reshape(1, 1, DIM)
         + (f2 * r)[:, :, None] * w[2].reshape(1, 1, DIM)
         + b_ref[...].reshape(1, 1, DIM))
    o_ref[...] = t.reshape(RBLK, DIM)


def _project(q0, q1, sx0, sx1, c0, c1, emb, W0, b0r):
    pk = pl.BlockSpec((PACK, DIM), lambda i: (i, 0))
    return pl.pallas_call(
        _k4_body,
        grid=(NBLK,),
        in_specs=[pk] * 6 + [pl.BlockSpec((2, 2), lambda i: (0, 0)),
                             pl.BlockSpec((3, DIM), lambda i: (0, 0)),
                             pl.BlockSpec((1, DIM), lambda i: (0, 0))],
        out_specs=pl.BlockSpec((RBLK, DIM), lambda i: (i, 0)),
        out_shape=jax.ShapeDtypeStruct((N, DIM), jnp.float32),
    )(q0, q1, sx0, sx1, c0, c1, emb, W0, b0r)


def kernel(significance, node_weight, edge_index, emb_table, W0, b0):
    ei = edge_index.astype(jnp.int32).reshape(2 * E)
    nwp = jnp.pad(node_weight.astype(jnp.float32), (0, NPAD - N)).reshape(ROWS, DIM)
    sgp = jnp.pad(significance.astype(jnp.float32), (0, NPAD - N)).reshape(ROWS, DIM)
    z1 = jnp.zeros((NPAD,), jnp.float32)
    zi1 = jnp.zeros((NPAD,), jnp.int32)
    ones_c = jnp.ones((CH,), jnp.float32)
    ones_c1 = jnp.ones((CH1,), jnp.float32)
    emb = emb_table.astype(jnp.float32)

    ho = _hist(ei, z1, ones_c1)
    wtbl = _table(ho[:NPAD].reshape(ROWS, DIM), ho[NPAD:].reshape(ROWS, DIM),
                  nwp, sgp)
    oq, osx, oc = _scatter(ei, wtbl.reshape(NPAD), z1, zi1, ones_c)
    qp = oq.reshape(NC, ROWS, DIM)
    sxp = osx.reshape(NC, ROWS, DIM)
    cp = oc.reshape(NC, ROWS, DIM)
    out = _project(qp[0], qp[1], sxp[0], sxp[1], cp[0], cp[1], emb,
                   W0.astype(jnp.float32),
                   b0.astype(jnp.float32).reshape(1, DIM))
    return out
